# Initial kernel scaffold; baseline (speedup 1.0000x reference)
import jax
import jax.numpy as jnp
from jax.experimental import pallas as pl


def kernel(h, x, edge_idx, W_e1, b_e1, W_e2, b_e2, W_c1, b_c1, W_c2, W_n1, b_n1, W_n2, b_n2):
    raise NotImplementedError("write your pallas kernel here")


# speedup vs baseline: 13.4935x; 13.4935x over previous
import functools

import jax
import jax.numpy as jnp
from jax import lax
from jax.experimental import pallas as pl
from jax.experimental.pallas import tpu as pltpu
from jax.experimental.pallas import tpu_sc as plsc

N = 10000
K = 32
H = 128
TCOLS = 256
E = N * K

BN = 200
EB = BN * K
NBLK = N // BN

NW = 32
PER_TILE = E // NW
CHUNK = 80
DEPTH = 5
GROUPS = PER_TILE // (CHUNK * DEPTH)


def _prep_body(h_ref, xp_ref, w1a_ref, w1b_ref, be1_ref, s_ref, t_ref):
    h = h_ref[...]
    s_ref[...] = (
        jnp.dot(h, w1a_ref[...], preferred_element_type=jnp.float32)
        + be1_ref[...]
    )
    bn = jnp.dot(h, w1b_ref[...], preferred_element_type=jnp.float32)
    t_ref[...] = jnp.concatenate([bn, xp_ref[...]], axis=1)


@functools.cache
def _make_gather():
    mesh = plsc.VectorSubcoreMesh(core_axis_name="c", subcore_axis_name="s")
    scratch = [pltpu.VMEM((PER_TILE,), jnp.int32)]
    scratch += [pltpu.VMEM((CHUNK, TCOLS), jnp.float32) for _ in range(DEPTH)]
    scratch += [pltpu.SemaphoreType.DMA for _ in range(2 * DEPTH)]

    @functools.partial(
        pl.kernel,
        mesh=mesh,
        out_type=jax.ShapeDtypeStruct((E, TCOLS), jnp.float32),
        scratch_types=scratch,
    )
    def gather_k(t_hbm, eidx_hbm, g_hbm, idx_v, *rest):
        bufs = rest[:DEPTH]
        gsems = rest[DEPTH:2 * DEPTH]
        ssems = rest[2 * DEPTH:]
        wid = lax.axis_index("s") * 2 + lax.axis_index("c")
        base = wid * PER_TILE
        pltpu.sync_copy(eidx_hbm.at[pl.ds(base, PER_TILE)], idx_v)

        def gstart(c, j):
            pltpu.async_copy(
                t_hbm.at[idx_v.at[pl.ds(c * CHUNK, CHUNK)]], bufs[j], gsems[j])

        def gwait(c, j):
            pltpu.make_async_copy(
                t_hbm.at[idx_v.at[pl.ds(c * CHUNK, CHUNK)]], bufs[j], gsems[j]
            ).wait()

        def sstart(c, j):
            pltpu.async_copy(
                bufs[j], g_hbm.at[pl.ds(base + c * CHUNK, CHUNK)], ssems[j])

        def swait(c, j):
            pltpu.make_async_copy(
                bufs[j], g_hbm.at[pl.ds(base + c * CHUNK, CHUNK)], ssems[j]
            ).wait()

        for j in range(DEPTH):
            gstart(j, j)

        def body(s, carry):
            c0 = s * DEPTH
            for j in range(DEPTH):
                gwait(c0 + j, j)
                sstart(c0 + j, j)
            for j in range(DEPTH):
                swait(c0 + j, j)

                @pl.when(s + 1 < GROUPS)
                def _():
                    gstart(c0 + DEPTH + j, j)

            return carry

        lax.fori_loop(0, GROUPS, body, 0)

    return gather_k


def _silu(v):
    return v * jax.nn.sigmoid(v)


def _edge_body(g_ref, s_ref, h_ref, xp_ref, wd_ref, we2_ref, be2_ref,
               wc1_ref, bc1_ref, wc2_ref, wn1h_ref, wn1m_ref, bn1_ref,
               wn2_ref, bn2_ref, hnew_ref, xnew_ref):
    g = g_ref[...]
    bn = g[:, :H]
    xj = g[:, H:H + 3]
    xi = xp_ref[...][:, :3]
    xib = jnp.broadcast_to(xi[:, None, :], (BN, K, 3)).reshape(EB, 3)
    diff = xib - xj
    sq = jnp.sum(diff * diff, axis=1, keepdims=True)
    sb = jnp.broadcast_to(
        s_ref[...][:, None, :], (BN, K, H)).reshape(EB, H)
    pre = sb + bn + sq * wd_ref[...]
    t1 = _silu(pre)
    m = _silu(jnp.dot(t1, we2_ref[...], preferred_element_type=jnp.float32)
              + be2_ref[...])
    c1 = _silu(jnp.dot(m, wc1_ref[...], preferred_element_type=jnp.float32)
               + bc1_ref[...])
    cw = jnp.dot(c1, wc2_ref[...], preferred_element_type=jnp.float32)[:, :1]
    m_i = jnp.sum(m.reshape(BN, K, H), axis=1)
    xupd = jnp.sum((diff * cw).reshape(BN, K, 3), axis=1) * (1.0 / K)
    xnew_ref[...] = xi + xupd
    h = h_ref[...]
    z = (jnp.dot(h, wn1h_ref[...], preferred_element_type=jnp.float32)
         + jnp.dot(m_i, wn1m_ref[...], preferred_element_type=jnp.float32)
         + bn1_ref[...])
    hnew_ref[...] = (
        jnp.dot(_silu(z), wn2_ref[...], preferred_element_type=jnp.float32)
        + bn2_ref[...] + h)


def _const_spec(shape):
    return pl.BlockSpec(shape, lambda i: (0, 0))


_prep = pl.pallas_call(
    _prep_body,
    out_shape=(
        jax.ShapeDtypeStruct((N, H), jnp.float32),
        jax.ShapeDtypeStruct((N, TCOLS), jnp.float32),
    ),
)

_edge = pl.pallas_call(
    _edge_body,
    grid=(NBLK,),
    in_specs=[
        pl.BlockSpec((EB, TCOLS), lambda i: (i, 0)),
        pl.BlockSpec((BN, H), lambda i: (i, 0)),
        pl.BlockSpec((BN, H), lambda i: (i, 0)),
        pl.BlockSpec((BN, 128), lambda i: (i, 0)),
        _const_spec((1, H)),
        _const_spec((H, H)),
        _const_spec((1, H)),
        _const_spec((H, H)),
        _const_spec((1, H)),
        _const_spec((H, 8)),
        _const_spec((H, H)),
        _const_spec((H, H)),
        _const_spec((1, H)),
        _const_spec((H, H)),
        _const_spec((1, H)),
    ],
    out_specs=(
        pl.BlockSpec((BN, H), lambda i: (i, 0)),
        pl.BlockSpec((BN, 3), lambda i: (i, 0)),
    ),
    out_shape=(
        jax.ShapeDtypeStruct((N, H), jnp.float32),
        jax.ShapeDtypeStruct((N, 3), jnp.float32),
    ),
)


def kernel(h, x, edge_idx, W_e1, b_e1, W_e2, b_e2, W_c1, b_c1, W_c2,
           W_n1, b_n1, W_n2, b_n2):
    h2 = h[0]
    x2 = x[0]
    eidx = edge_idx[0].reshape(E)
    xp = jnp.concatenate([x2, jnp.zeros((N, 125), jnp.float32)], axis=1)
    w1a = W_e1[:H]
    w1b = W_e1[H:2 * H]
    wd = W_e1[2 * H:2 * H + 1]
    be1 = b_e1.reshape(1, H)
    be2 = b_e2.reshape(1, H)
    bc1 = b_c1.reshape(1, H)
    bn1 = b_n1.reshape(1, H)
    bn2 = b_n2.reshape(1, H)
    wc2p = jnp.pad(W_c2, ((0, 0), (0, 7)))
    wn1h = W_n1[:H]
    wn1m = W_n1[H:]

    s_arr, t_arr = _prep(h2, xp, w1a, w1b, be1)
    g_arr = _make_gather()(t_arr, eidx)
    h_new, x_new = _edge(g_arr, s_arr, h2, xp, wd, W_e2, be2, W_c1, bc1,
                         wc2p, wn1h, wn1m, bn1, W_n2, bn2)
    return (h_new[None], x_new[None])


# speedup vs baseline: 14.8868x; 1.1033x over previous
import functools

import jax
import jax.numpy as jnp
from jax import lax
from jax.experimental import pallas as pl
from jax.experimental.pallas import tpu as pltpu
from jax.experimental.pallas import tpu_sc as plsc

N = 10000
K = 32
H = 128
HH = H // 2
TCOLS = 128
E = N * K

BN = 200
EB = BN * K
NBLK = N // BN

NW = 32
PER_TILE = E // NW
CHUNK = 80
DEPTH = 5
GROUPS = PER_TILE // (CHUNK * DEPTH)

def _dotbf(a, b):
    return jnp.dot(a.astype(jnp.bfloat16), b.astype(jnp.bfloat16),
                   preferred_element_type=jnp.float32)


def _silu(v):
    return v * jax.nn.sigmoid(v)


def _prep_body(h_ref, x_ref, w1a_ref, w1b_ref, be1_ref, s_ref, t_ref):
    h = h_ref[0]
    s_ref[0] = (
        jnp.dot(h, w1a_ref[...], preferred_element_type=jnp.float32)
        + be1_ref[...]
    )
    bn = _dotbf(h, w1b_ref[...])
    bnb = lax.bitcast_convert_type(bn, jnp.int32)
    lo16 = lax.shift_right_logical(bnb[:, :HH] + 0x8000, 16)
    hi16 = (bnb[:, HH:] + 0x8000) & (-65536)
    t_ref[:, :HH] = lax.bitcast_convert_type(hi16 | lo16, jnp.float32)
    t_ref[:, HH:HH + 3] = x_ref[0]
    t_ref[:, HH + 3:] = jnp.zeros((N, TCOLS - HH - 3), jnp.float32)


@functools.cache
def _make_gather():
    mesh = plsc.VectorSubcoreMesh(core_axis_name="c", subcore_axis_name="s")
    scratch = [pltpu.VMEM((PER_TILE,), jnp.int32)]
    scratch += [pltpu.VMEM((CHUNK, TCOLS), jnp.float32) for _ in range(DEPTH)]
    scratch += [pltpu.SemaphoreType.DMA for _ in range(2 * DEPTH)]

    @functools.partial(
        pl.kernel,
        mesh=mesh,
        out_type=jax.ShapeDtypeStruct((E, TCOLS), jnp.float32),
        scratch_types=scratch,
    )
    def gather_k(t_hbm, eidx_hbm, g_hbm, idx_v, *rest):
        bufs = rest[:DEPTH]
        gsems = rest[DEPTH:2 * DEPTH]
        ssems = rest[2 * DEPTH:]
        wid = lax.axis_index("s") * 2 + lax.axis_index("c")
        base = wid * PER_TILE
        pltpu.sync_copy(eidx_hbm.at[pl.ds(base, PER_TILE)], idx_v)

        def gstart(c, j):
            pltpu.async_copy(
                t_hbm.at[idx_v.at[pl.ds(c * CHUNK, CHUNK)]], bufs[j], gsems[j])

        def gwait(c, j):
            pltpu.make_async_copy(
                t_hbm.at[idx_v.at[pl.ds(c * CHUNK, CHUNK)]], bufs[j], gsems[j]
            ).wait()

        def sstart(c, j):
            pltpu.async_copy(
                bufs[j], g_hbm.at[pl.ds(base + c * CHUNK, CHUNK)], ssems[j])

        def swait(c, j):
            pltpu.make_async_copy(
                bufs[j], g_hbm.at[pl.ds(base + c * CHUNK, CHUNK)], ssems[j]
            ).wait()

        for j in range(DEPTH):
            gstart(j, j)

        def body(s, carry):
            c0 = s * DEPTH
            for j in range(DEPTH):
                gwait(c0 + j, j)
                sstart(c0 + j, j)
            for j in range(DEPTH):
                swait(c0 + j, j)

                @pl.when(s + 1 < GROUPS)
                def _():
                    gstart(c0 + DEPTH + j, j)

            return carry

        lax.fori_loop(0, GROUPS, body, 0)

    return gather_k


def _edge_body(g_ref, s_ref, h_ref, x_ref, wd_ref, we2_ref, be2_ref,
               wc1_ref, bc1_ref, wc2_ref, wn1h_ref, wn1m_ref, bn1_ref,
               wn2_ref, bn2_ref, hnew_ref, xnew_ref):
    g = g_ref[...]
    gw = lax.bitcast_convert_type(g[:, :HH], jnp.int32)
    bn_lo = lax.bitcast_convert_type(lax.shift_left(gw, 16), jnp.float32)
    bn_hi = lax.bitcast_convert_type(gw & (-65536), jnp.float32)
    bn = jnp.concatenate([bn_lo, bn_hi], axis=1)
    xj = g[:, HH:HH + 3]
    xi = x_ref[0]
    xib = jnp.broadcast_to(xi[:, None, :], (BN, K, 3)).reshape(EB, 3)
    diff = xib - xj
    sq = jnp.sum(diff * diff, axis=1, keepdims=True)
    sb = jnp.broadcast_to(
        s_ref[0][:, None, :], (BN, K, H)).reshape(EB, H)
    pre = sb + bn + sq * wd_ref[...]
    t1 = _silu(pre)
    m = _silu(_dotbf(t1, we2_ref[...]) + be2_ref[...])
    c1 = _silu(_dotbf(m, wc1_ref[...]) + bc1_ref[...])
    cw = _dotbf(c1, wc2_ref[...])[:, :1]
    m_i = jnp.sum(m.reshape(BN, K, H), axis=1)
    xupd = jnp.sum((diff * cw).reshape(BN, K, 3), axis=1) * (1.0 / K)
    xnew_ref[0] = xi + xupd
    h = h_ref[0]
    z = (_dotbf(h, wn1h_ref[...]) + _dotbf(m_i, wn1m_ref[...])
         + bn1_ref[...])
    hnew_ref[0] = _dotbf(_silu(z), wn2_ref[...]) + bn2_ref[...] + h


def _const_spec(shape):
    return pl.BlockSpec(shape, lambda i: tuple(0 for _ in shape))


_prep = pl.pallas_call(
    _prep_body,
    out_shape=(
        jax.ShapeDtypeStruct((1, N, H), jnp.float32),
        jax.ShapeDtypeStruct((N, TCOLS), jnp.float32),
    ),
)

_edge = pl.pallas_call(
    _edge_body,
    grid=(NBLK,),
    in_specs=[
        pl.BlockSpec((EB, TCOLS), lambda i: (i, 0)),
        pl.BlockSpec((1, BN, H), lambda i: (0, i, 0)),
        pl.BlockSpec((1, BN, H), lambda i: (0, i, 0)),
        pl.BlockSpec((1, BN, 3), lambda i: (0, i, 0)),
        _const_spec((1, H)),
        _const_spec((H, H)),
        _const_spec((1, H)),
        _const_spec((H, H)),
        _const_spec((1, H)),
        _const_spec((H, 8)),
        _const_spec((H, H)),
        _const_spec((H, H)),
        _const_spec((1, H)),
        _const_spec((H, H)),
        _const_spec((1, H)),
    ],
    out_specs=(
        pl.BlockSpec((1, BN, H), lambda i: (0, i, 0)),
        pl.BlockSpec((1, BN, 3), lambda i: (0, i, 0)),
    ),
    out_shape=(
        jax.ShapeDtypeStruct((1, N, H), jnp.float32),
        jax.ShapeDtypeStruct((1, N, 3), jnp.float32),
    ),
)


def kernel(h, x, edge_idx, W_e1, b_e1, W_e2, b_e2, W_c1, b_c1, W_c2,
           W_n1, b_n1, W_n2, b_n2):
    eidx = edge_idx.reshape(E)
    w1a = W_e1[:H]
    w1b = W_e1[H:2 * H]
    wd = W_e1[2 * H:2 * H + 1]
    be1 = b_e1.reshape(1, H)
    be2 = b_e2.reshape(1, H)
    bc1 = b_c1.reshape(1, H)
    bn1 = b_n1.reshape(1, H)
    bn2 = b_n2.reshape(1, H)
    wc2p = jnp.pad(W_c2, ((0, 0), (0, 7)))
    wn1h = W_n1[:H]
    wn1m = W_n1[H:]

    s_arr, t_arr = _prep(h, x, w1a, w1b, be1)
    g_arr = _make_gather()(t_arr, eidx)
    h_new, x_new = _edge(g_arr, s_arr, h, x, wd, W_e2, be2, W_c1, bc1,
                         wc2p, wn1h, wn1m, bn1, W_n2, bn2)
    return (h_new, x_new)


# speedup vs baseline: 19.1959x; 1.2895x over previous
import functools

import jax
import jax.numpy as jnp
from jax import lax
from jax.experimental import pallas as pl
from jax.experimental.pallas import tpu as pltpu
from jax.experimental.pallas import tpu_sc as plsc

N = 10000
K = 32
H = 128
HH = H // 2
TCOLS = 128
E = N * K

BN = 200
EB = BN * K
NBLK = N // BN

NCH = 2
E2 = E // NCH
N2 = N // NCH
NBLK2 = NBLK // NCH

NW = 32
PER_TILE = E2 // NW
CHUNK = 40
DEPTH = 5
GROUPS = PER_TILE // (CHUNK * DEPTH)

def _dotbf(a, b):
    return jnp.dot(a.astype(jnp.bfloat16), b.astype(jnp.bfloat16),
                   preferred_element_type=jnp.float32)


def _silu(v):
    hv = 0.5 * v
    return hv * jnp.tanh(hv) + hv


def _silu_bf(v):
    hv = jnp.bfloat16(0.5) * v.astype(jnp.bfloat16)
    return hv * jnp.tanh(hv) + hv


def _prep_body(h_ref, x_ref, w1a_ref, w1b_ref, be1_ref, s_ref, t_ref):
    h = h_ref[0]
    s_ref[0] = (
        jnp.dot(h, w1a_ref[...], preferred_element_type=jnp.float32)
        + be1_ref[...]
    )
    bn = _dotbf(h, w1b_ref[...])
    bnb = lax.bitcast_convert_type(bn, jnp.int32)
    lo16 = lax.shift_right_logical(bnb[:, :HH] + 0x8000, 16)
    hi16 = (bnb[:, HH:] + 0x8000) & (-65536)
    t_ref[:, :HH] = lax.bitcast_convert_type(hi16 | lo16, jnp.float32)
    t_ref[:, HH:HH + 3] = x_ref[0]
    t_ref[:, HH + 3:] = jnp.zeros((N, TCOLS - HH - 3), jnp.float32)


@functools.cache
def _make_gather():
    mesh = plsc.VectorSubcoreMesh(core_axis_name="c", subcore_axis_name="s")
    scratch = [pltpu.VMEM((PER_TILE,), jnp.int32)]
    scratch += [pltpu.VMEM((CHUNK, TCOLS), jnp.float32) for _ in range(DEPTH)]
    scratch += [pltpu.SemaphoreType.DMA for _ in range(2 * DEPTH)]

    @functools.partial(
        pl.kernel,
        mesh=mesh,
        out_type=jax.ShapeDtypeStruct((E2, TCOLS), jnp.float32),
        scratch_types=scratch,
    )
    def gather_k(t_hbm, eidx_hbm, g_hbm, idx_v, *rest):
        bufs = rest[:DEPTH]
        gsems = rest[DEPTH:2 * DEPTH]
        ssems = rest[2 * DEPTH:]
        wid = lax.axis_index("s") * 2 + lax.axis_index("c")
        base = wid * PER_TILE
        pltpu.sync_copy(eidx_hbm.at[pl.ds(base, PER_TILE)], idx_v)

        def gstart(c, j):
            pltpu.async_copy(
                t_hbm.at[idx_v.at[pl.ds(c * CHUNK, CHUNK)]], bufs[j], gsems[j])

        def gwait(c, j):
            pltpu.make_async_copy(
                t_hbm.at[idx_v.at[pl.ds(c * CHUNK, CHUNK)]], bufs[j], gsems[j]
            ).wait()

        def sstart(c, j):
            pltpu.async_copy(
                bufs[j], g_hbm.at[pl.ds(base + c * CHUNK, CHUNK)], ssems[j])

        def swait(c, j):
            pltpu.make_async_copy(
                bufs[j], g_hbm.at[pl.ds(base + c * CHUNK, CHUNK)], ssems[j]
            ).wait()

        for j in range(DEPTH):
            gstart(j, j)

        def body(s, carry):
            c0 = s * DEPTH
            for j in range(DEPTH):
                gwait(c0 + j, j)
                sstart(c0 + j, j)
            for j in range(DEPTH):
                swait(c0 + j, j)

                @pl.when(s + 1 < GROUPS)
                def _():
                    gstart(c0 + DEPTH + j, j)

            return carry

        lax.fori_loop(0, GROUPS, body, 0)

    return gather_k


def _edge_body(g_ref, s_ref, h_ref, x_ref, wd_ref, we2_ref, be2_ref,
               wc1_ref, bc1_ref, wc2_ref, wn1h_ref, wn1m_ref, bn1_ref,
               wn2_ref, bn2_ref, hnew_ref, xnew_ref):
    g = g_ref[...]
    gw = lax.bitcast_convert_type(g[:, :HH], jnp.int32)
    bn_lo = lax.bitcast_convert_type(lax.shift_left(gw, 16), jnp.float32)
    bn_hi = lax.bitcast_convert_type(gw & (-65536), jnp.float32)
    bn = jnp.concatenate([bn_lo, bn_hi], axis=1)
    xj = g[:, HH:HH + 3]
    xi = x_ref[0]
    xib = jnp.broadcast_to(xi[:, None, :], (BN, K, 3)).reshape(EB, 3)
    diff = xib - xj
    sq = jnp.sum(diff * diff, axis=1, keepdims=True)
    sb = jnp.broadcast_to(
        s_ref[0][:, None, :], (BN, K, H)).reshape(EB, H)
    pre = sb + bn + sq * wd_ref[...]
    t1 = _silu_bf(pre)
    m = _silu_bf(_dotbf(t1, we2_ref[...]) + be2_ref[...])
    c1 = _silu_bf(_dotbf(m, wc1_ref[...]) + bc1_ref[...])
    cw = _dotbf(c1, wc2_ref[...])[:, :1]
    m_i = jnp.sum(m.reshape(BN, K, H), axis=1,
                  dtype=jnp.float32)
    xupd = jnp.sum((diff * cw).reshape(BN, K, 3), axis=1) * (1.0 / K)
    xnew_ref[0] = xi + xupd
    h = h_ref[0]
    z = (_dotbf(h, wn1h_ref[...]) + _dotbf(m_i, wn1m_ref[...])
         + bn1_ref[...])
    hnew_ref[0] = _dotbf(_silu(z), wn2_ref[...]) + bn2_ref[...] + h


def _const_spec(shape):
    return pl.BlockSpec(shape, lambda i: tuple(0 for _ in shape))


_prep = pl.pallas_call(
    _prep_body,
    out_shape=(
        jax.ShapeDtypeStruct((1, N, H), jnp.float32),
        jax.ShapeDtypeStruct((N, TCOLS), jnp.float32),
    ),
)

@functools.cache
def _make_edge(off):
    return pl.pallas_call(
        _edge_body,
        grid=(NBLK2,),
        in_specs=[
            pl.BlockSpec((EB, TCOLS), lambda i: (i, 0)),
            pl.BlockSpec((1, BN, H), lambda i: (0, i + off, 0)),
            pl.BlockSpec((1, BN, H), lambda i: (0, i + off, 0)),
            pl.BlockSpec((1, BN, 3), lambda i: (0, i + off, 0)),
            _const_spec((1, H)),
            _const_spec((H, H)),
            _const_spec((1, H)),
            _const_spec((H, H)),
            _const_spec((1, H)),
            _const_spec((H, 8)),
            _const_spec((H, H)),
            _const_spec((H, H)),
            _const_spec((1, H)),
            _const_spec((H, H)),
            _const_spec((1, H)),
        ],
        out_specs=(
            pl.BlockSpec((1, BN, H), lambda i: (0, i, 0)),
            pl.BlockSpec((1, BN, 3), lambda i: (0, i, 0)),
        ),
        out_shape=(
            jax.ShapeDtypeStruct((1, N2, H), jnp.float32),
            jax.ShapeDtypeStruct((1, N2, 3), jnp.float32),
        ),
    )


def kernel(h, x, edge_idx, W_e1, b_e1, W_e2, b_e2, W_c1, b_c1, W_c2,
           W_n1, b_n1, W_n2, b_n2):
    eidx = edge_idx.reshape(E)
    w1a = W_e1[:H]
    w1b = W_e1[H:2 * H]
    wd = W_e1[2 * H:2 * H + 1]
    be1 = b_e1.reshape(1, H)
    be2 = b_e2.reshape(1, H)
    bc1 = b_c1.reshape(1, H)
    bn1 = b_n1.reshape(1, H)
    bn2 = b_n2.reshape(1, H)
    wc2p = jnp.pad(W_c2, ((0, 0), (0, 7)))
    wn1h = W_n1[:H]
    wn1m = W_n1[H:]

    s_arr, t_arr = _prep(h, x, w1a, w1b, be1)
    gather = _make_gather()
    weights = (wd, W_e2, be2, W_c1, bc1, wc2p, wn1h, wn1m, bn1, W_n2, bn2)
    h_halves, x_halves = [], []
    for c in range(NCH):
        eidx_c = lax.slice(eidx, (c * E2,), ((c + 1) * E2,))
        g_c = gather(t_arr, eidx_c)
        hn, xn = _make_edge(c * NBLK2)(g_c, s_arr, h, x, *weights)
        h_halves.append(hn)
        x_halves.append(xn)
    h_new = jnp.concatenate(h_halves, axis=1)
    x_new = jnp.concatenate(x_halves, axis=1)
    return (h_new, x_new)
